# baseline (speedup 1.0000x reference)
import functools

import jax
import jax.numpy as jnp
from jax import lax
from jax.experimental import pallas as pl
from jax.experimental.pallas import tpu as pltpu
from jax.experimental.pallas import tpu_sc as plsc

ROWS = 64
D = 64
BASE = 7
STRIDE = 15625
L = 16
NWORK = ROWS // L


def kernel(a):
    info = plsc.get_sparse_core_info()
    nc = info.num_cores

    mesh = plsc.VectorSubcoreMesh(core_axis_name="c", subcore_axis_name="s")

    @functools.partial(
        pl.kernel,
        mesh=mesh,
        out_type=jax.ShapeDtypeStruct((ROWS, D), jnp.float32),
        scratch_types=[
            pltpu.VMEM((L, D), jnp.float32),
            pltpu.SemaphoreType.DMA,
        ],
        compiler_params=pltpu.CompilerParams(use_tc_tiling_on_sc=False),
    )
    def sc_gather(table_hbm, out_hbm, rows_v, sem):
        wid = lax.axis_index("s") * nc + lax.axis_index("c")

        @pl.when(wid < NWORK)
        def _():
            idx = lax.iota(jnp.int32, L) * STRIDE + (BASE + wid * (L * STRIDE))
            pltpu.async_copy(table_hbm.at[idx], rows_v, sem).wait()
            for i in range(L):
                for j in range(D // L):
                    sl = pl.ds(j * L, L)
                    rows_v[i, sl] = rows_v[i, sl] * 2.0
            pltpu.sync_copy(rows_v, out_hbm.at[pl.ds(wid * L, L)])

    return sc_gather(a)


# speedup vs baseline: 1.7391x; 1.7391x over previous
import functools

import jax
import jax.numpy as jnp
from jax import lax
from jax.experimental import pallas as pl
from jax.experimental.pallas import tpu as pltpu
from jax.experimental.pallas import tpu_sc as plsc

ROWS = 64
D = 64
BASE = 7
STRIDE = 15625
L = 16
NWORK = ROWS // L


def kernel(a):
    info = plsc.get_sparse_core_info()
    nc = info.num_cores

    mesh = plsc.VectorSubcoreMesh(core_axis_name="c", subcore_axis_name="s")

    @functools.partial(
        pl.kernel,
        mesh=mesh,
        out_type=jax.ShapeDtypeStruct((ROWS, D), jnp.float32),
        scratch_types=[
            pltpu.VMEM((L, 8, D), jnp.float32),
            pltpu.VMEM((L, D), jnp.float32),
            pltpu.SemaphoreType.DMA,
        ],
    )
    def sc_gather(table_hbm, out_hbm, tiles_v, rows_v, sem):
        wid = lax.axis_index("s") * nc + lax.axis_index("c")

        @pl.when(wid < NWORK)
        def _():
            base0 = wid * (L * STRIDE)
            copies = []
            for i in range(L):
                sub = (BASE + i * STRIDE) % 8
                start = base0 + BASE + i * STRIDE - sub
                copies.append(
                    pltpu.async_copy(
                        table_hbm.at[pl.ds(start, 8), :], tiles_v.at[i], sem
                    )
                )
            for cp in copies:
                cp.wait()
            for i in range(L):
                sub = (BASE + i * STRIDE) % 8
                for j in range(D // L):
                    sl = pl.ds(j * L, L)
                    rows_v[i, sl] = tiles_v[i, sub, sl] * 2.0
            pltpu.sync_copy(rows_v, out_hbm.at[pl.ds(wid * L, L)])

    return sc_gather(a)


# speedup vs baseline: 26.8036x; 15.4122x over previous
import functools

import jax
import jax.numpy as jnp
from jax import lax
from jax.experimental import pallas as pl
from jax.experimental.pallas import tpu as pltpu
from jax.experimental.pallas import tpu_sc as plsc

ROWS = 64
D = 64
BASE = 7
STRIDE = 15625
L = 16
NWORK = 8
KPW = ROWS // NWORK


def kernel(a):
    info = plsc.get_sparse_core_info()
    nc = info.num_cores

    mesh = plsc.VectorSubcoreMesh(core_axis_name="c", subcore_axis_name="s")

    @functools.partial(
        pl.kernel,
        mesh=mesh,
        out_type=jax.ShapeDtypeStruct((ROWS, D), jnp.float32),
        scratch_types=[
            pltpu.VMEM((KPW, D, 128), jnp.float32),
            pltpu.VMEM((KPW, D), jnp.float32),
            pltpu.SemaphoreType.DMA,
        ],
        compiler_params=pltpu.CompilerParams(needs_layout_passes=False),
    )
    def sc_gather(att_hbm, out_hbm, blocks_v, rows_v, sem):
        wid = lax.axis_index("s") * nc + lax.axis_index("c")

        @pl.when(wid < NWORK)
        def _():
            base0 = BASE + wid * (KPW * STRIDE)
            copies = []
            for i in range(KPW):
                idx = base0 + i * STRIDE
                q0 = pl.multiple_of((idx // 128) * 128, 128)
                copies.append(
                    pltpu.async_copy(
                        att_hbm.at[:, pl.ds(q0, 128)], blocks_v.at[i], sem
                    )
                )
            for cp in copies:
                cp.wait()
            for i in range(KPW):
                idx = base0 + i * STRIDE
                col = jnp.full((L,), lax.rem(idx, 128), dtype=jnp.int32)
                for g in range(D // L):
                    row = lax.iota(jnp.int32, L) + g * L
                    vals = plsc.load_gather(blocks_v.at[i], [row, col])
                    rows_v[i, pl.ds(g * L, L)] = vals * 2.0
            pltpu.sync_copy(rows_v, out_hbm.at[pl.ds(wid * KPW, KPW)])

    return sc_gather(a.T)
